# baseline (speedup 1.0000x reference)
import functools

import jax
import jax.numpy as jnp
from jax import lax
from jax.experimental import pallas as pl
from jax.experimental.pallas import tpu as pltpu
from jax.experimental.pallas import tpu_sc as plsc

B, C, H, W = 16, 3, 512, 512
HW = H * W
W2 = W // 2
NW = 32
PIX_PER_TILE = B * HW // NW
P = 2048
NCHUNK = PIX_PER_TILE // P
NG = P // 16
NIDX = 4 * P
NDMA = NIDX // 128


def _make_warp():
    mesh = plsc.VectorSubcoreMesh(core_axis_name="c", subcore_axis_name="s")

    @functools.partial(
        pl.kernel,
        mesh=mesh,
        out_type=jax.ShapeDtypeStruct((B * C, HW), jnp.float32),
        compiler_params=pltpu.CompilerParams(
            needs_layout_passes=False, use_tc_tiling_on_sc=False),
        scratch_types=[
            pltpu.VMEM((P,), jnp.float32),
            pltpu.VMEM((P,), jnp.float32),
            pltpu.VMEM((P,), jnp.float32),
            pltpu.VMEM((P,), jnp.float32),
            pltpu.VMEM((P,), jnp.int32),
            pltpu.VMEM((NIDX,), jnp.int32),
            pltpu.VMEM((NIDX, 8), jnp.float32),
            pltpu.VMEM((P,), jnp.float32),
            pltpu.VMEM((P,), jnp.float32),
            pltpu.VMEM((P,), jnp.float32),
            pltpu.SemaphoreType.DMA,
        ],
    )
    def warp(table, flow, out, fx_v, fy_v, wx_v, wy_v, ex_v, idx_v, g_v,
             o0, o1, o2, sem):
        wid = lax.axis_index("s") * 2 + lax.axis_index("c")
        b = wid // 2
        half = wid % 2

        def chunk_body(ch, carry):
            off = half * PIX_PER_TILE + ch * P
            row0 = off // W
            pltpu.sync_copy(flow.at[2 * b, pl.ds(off, P)], fx_v)
            pltpu.sync_copy(flow.at[2 * b + 1, pl.ds(off, P)], fy_v)

            def group_a(i, c2):
                r = row0 + i // (W // 16)
                jb = (i % (W // 16)) * 16
                lane = lax.iota(jnp.int32, 16)
                jf = (jb + lane).astype(jnp.float32)
                fx = fx_v[pl.ds(i * 16, 16)]
                fy = fy_v[pl.ds(i * 16, 16)]
                x = jnp.clip(jf - fx, 0.0, float(W - 1))
                y = jnp.clip(r.astype(jnp.float32) - fy, 0.0, float(H - 1))
                x0 = jnp.minimum(x.astype(jnp.int32), W - 2)
                y0 = jnp.minimum(y.astype(jnp.int32), H - 2)
                wx_v[pl.ds(i * 16, 16)] = x - x0.astype(jnp.float32)
                wy_v[pl.ds(i * 16, 16)] = y - y0.astype(jnp.float32)
                ex_v[pl.ds(i * 16, 16)] = (x0 & 1) << 2
                rt = (b * H + y0) * W2 + (x0 >> 1)
                rb = rt + W2
                pos = 4 * (i * 16 + lane)
                plsc.store_scatter(idx_v, [pos], rt)
                plsc.store_scatter(idx_v, [pos + 1], rt + 1)
                plsc.store_scatter(idx_v, [pos + 2], rb)
                plsc.store_scatter(idx_v, [pos + 3], rb + 1)
                return c2

            lax.fori_loop(0, NG, group_a, 0)

            def dma_body(k, c2):
                handles = []
                for j in range(8):
                    d = k * 8 + j
                    handles.append(pltpu.async_copy(
                        table.at[idx_v.at[pl.ds(d * 128, 128)]],
                        g_v.at[pl.ds(d * 128, 128)],
                        sem))
                for h in handles:
                    h.wait()
                return c2

            lax.fori_loop(0, NDMA // 8, dma_body, 0)

            def group_b(i, c2):
                lane = lax.iota(jnp.int32, 16)
                wx = wx_v[pl.ds(i * 16, 16)]
                wy = wy_v[pl.ds(i * 16, 16)]
                e = ex_v[pl.ds(i * 16, 16)]
                wxm = 1.0 - wx
                wym = 1.0 - wy
                p4 = 4 * (i * 16 + lane)
                res = []
                for c in range(3):
                    t0 = e + c
                    t4 = t0 + 4
                    ro = t4 >> 3
                    co = t4 & 7
                    v00 = plsc.load_gather(g_v, [p4, t0])
                    v01 = plsc.load_gather(g_v, [p4 + ro, co])
                    v10 = plsc.load_gather(g_v, [p4 + 2, t0])
                    v11 = plsc.load_gather(g_v, [p4 + 2 + ro, co])
                    res.append((v00 * wxm + v01 * wx) * wym
                               + (v10 * wxm + v11 * wx) * wy)
                o0[pl.ds(i * 16, 16)] = res[0]
                o1[pl.ds(i * 16, 16)] = res[1]
                o2[pl.ds(i * 16, 16)] = res[2]
                return c2

            lax.fori_loop(0, NG, group_b, 0)
            pltpu.sync_copy(o0, out.at[3 * b, pl.ds(off, P)])
            pltpu.sync_copy(o1, out.at[3 * b + 1, pl.ds(off, P)])
            pltpu.sync_copy(o2, out.at[3 * b + 2, pl.ds(off, P)])
            return carry

        lax.fori_loop(0, NCHUNK, chunk_body, 0)

    return warp


_warp = _make_warp()


@jax.jit
def kernel(image, flow):
    t = jnp.pad(image, ((0, 0), (0, 1), (0, 0), (0, 0)))
    table = jnp.transpose(t, (0, 2, 3, 1)).reshape(B * HW // 2, 8)
    table = jnp.pad(table, ((0, 8), (0, 0)))
    flow2 = flow.reshape(B * 2, HW)
    out = _warp(table, flow2)
    return out.reshape(B, C, H, W)


# speedup vs baseline: 4.5838x; 4.5838x over previous
import functools

import jax
import jax.numpy as jnp
from jax import lax
from jax.experimental import pallas as pl
from jax.experimental.pallas import tpu as pltpu
from jax.experimental.pallas import tpu_sc as plsc

B, C, H, W = 16, 3, 512, 512
HW = H * W
W2 = W // 2
NW = 32
PIX_PER_TILE = B * HW // NW
TROWS = B * HW // 2 + 8

P = 2048
NCHUNK = PIX_PER_TILE // P
NG = P // 16
NIDX = 4 * P
NDMA = NIDX // 128

IR = 8
IPIX = IR * W
ICHUNK = 256 // IR

_MESH = plsc.VectorSubcoreMesh(core_axis_name="c", subcore_axis_name="s")
_CP = pltpu.CompilerParams(
    needs_layout_passes=False, use_tc_tiling_on_sc=False)


def _make_interleave():
    @functools.partial(
        pl.kernel,
        mesh=_MESH,
        out_type=jax.ShapeDtypeStruct((TROWS, 8), jnp.float32),
        compiler_params=_CP,
        scratch_types=[
            pltpu.VMEM((IR, W), jnp.float32),
            pltpu.VMEM((IR, W), jnp.float32),
            pltpu.VMEM((IR, W), jnp.float32),
            pltpu.VMEM((IPIX // 2, 8), jnp.float32),
        ],
    )
    def interleave(img, table, i0, i1, i2, buf):
        wid = lax.axis_index("s") * 2 + lax.axis_index("c")
        b = wid // 2
        half = wid % 2

        def chunk_body(ch, carry):
            r0 = half * 256 + ch * IR
            pltpu.sync_copy(img.at[3 * b, pl.ds(r0, IR), :], i0)
            pltpu.sync_copy(img.at[3 * b + 1, pl.ds(r0, IR), :], i1)
            pltpu.sync_copy(img.at[3 * b + 2, pl.ds(r0, IR), :], i2)

            def group(i, c2):
                rl = i // (W // 16)
                jb = (i % (W // 16)) * 16
                lane = lax.iota(jnp.int32, 16)
                px = jb + lane
                blk = rl * W2 + (px >> 1)
                col = (px & 1) << 2
                plsc.store_scatter(buf, [blk, col], i0[rl, pl.ds(jb, 16)])
                plsc.store_scatter(buf, [blk, col + 1], i1[rl, pl.ds(jb, 16)])
                plsc.store_scatter(buf, [blk, col + 2], i2[rl, pl.ds(jb, 16)])
                return c2

            lax.fori_loop(0, IPIX // 16, group, 0)
            pltpu.sync_copy(buf, table.at[pl.ds((b * H + r0) * W2, IPIX // 2)])
            return carry

        lax.fori_loop(0, ICHUNK, chunk_body, 0)

    return interleave


def _make_warp():
    @functools.partial(
        pl.kernel,
        mesh=_MESH,
        out_type=jax.ShapeDtypeStruct((B * C, HW), jnp.float32),
        compiler_params=_CP,
        scratch_types=[
            pltpu.VMEM((P,), jnp.float32),
            pltpu.VMEM((P,), jnp.float32),
            pltpu.VMEM((P,), jnp.float32),
            pltpu.VMEM((P,), jnp.float32),
            pltpu.VMEM((P,), jnp.int32),
            pltpu.VMEM((NIDX,), jnp.int32),
            pltpu.VMEM((NIDX, 8), jnp.float32),
            pltpu.VMEM((P,), jnp.float32),
            pltpu.VMEM((P,), jnp.float32),
            pltpu.VMEM((P,), jnp.float32),
            pltpu.SemaphoreType.DMA,
        ],
    )
    def warp(table, flow, out, fx_v, fy_v, wx_v, wy_v, ex_v, idx_v, g_v,
             o0, o1, o2, sem):
        wid = lax.axis_index("s") * 2 + lax.axis_index("c")
        b = wid // 2
        half = wid % 2

        def chunk_body(ch, carry):
            off = half * PIX_PER_TILE + ch * P
            row0 = off // W
            pltpu.sync_copy(flow.at[2 * b, pl.ds(off, P)], fx_v)
            pltpu.sync_copy(flow.at[2 * b + 1, pl.ds(off, P)], fy_v)

            def group_a(i, c2):
                r = row0 + i // (W // 16)
                jb = (i % (W // 16)) * 16
                lane = lax.iota(jnp.int32, 16)
                jf = (jb + lane).astype(jnp.float32)
                fx = fx_v[pl.ds(i * 16, 16)]
                fy = fy_v[pl.ds(i * 16, 16)]
                x = jnp.clip(jf - fx, 0.0, float(W - 1))
                y = jnp.clip(r.astype(jnp.float32) - fy, 0.0, float(H - 1))
                x0 = jnp.minimum(x.astype(jnp.int32), W - 2)
                y0 = jnp.minimum(y.astype(jnp.int32), H - 2)
                wx_v[pl.ds(i * 16, 16)] = x - x0.astype(jnp.float32)
                wy_v[pl.ds(i * 16, 16)] = y - y0.astype(jnp.float32)
                ex_v[pl.ds(i * 16, 16)] = (x0 & 1) << 2
                rt = (b * H + y0) * W2 + (x0 >> 1)
                rb = rt + W2
                pos = 4 * (i * 16 + lane)
                plsc.store_scatter(idx_v, [pos], rt)
                plsc.store_scatter(idx_v, [pos + 1], rt + 1)
                plsc.store_scatter(idx_v, [pos + 2], rb)
                plsc.store_scatter(idx_v, [pos + 3], rb + 1)
                return c2

            lax.fori_loop(0, NG, group_a, 0)

            def dma_body(k, c2):
                handles = []
                for j in range(8):
                    d = k * 8 + j
                    handles.append(pltpu.async_copy(
                        table.at[idx_v.at[pl.ds(d * 128, 128)]],
                        g_v.at[pl.ds(d * 128, 128)],
                        sem))
                for h in handles:
                    h.wait()
                return c2

            lax.fori_loop(0, NDMA // 8, dma_body, 0)

            def group_b(i, c2):
                lane = lax.iota(jnp.int32, 16)
                wx = wx_v[pl.ds(i * 16, 16)]
                wy = wy_v[pl.ds(i * 16, 16)]
                e = ex_v[pl.ds(i * 16, 16)]
                wxm = 1.0 - wx
                wym = 1.0 - wy
                p4 = 4 * (i * 16 + lane)
                res = []
                for c in range(3):
                    t0 = e + c
                    t4 = t0 + 4
                    ro = t4 >> 3
                    co = t4 & 7
                    v00 = plsc.load_gather(g_v, [p4, t0])
                    v01 = plsc.load_gather(g_v, [p4 + ro, co])
                    v10 = plsc.load_gather(g_v, [p4 + 2, t0])
                    v11 = plsc.load_gather(g_v, [p4 + 2 + ro, co])
                    res.append((v00 * wxm + v01 * wx) * wym
                               + (v10 * wxm + v11 * wx) * wy)
                o0[pl.ds(i * 16, 16)] = res[0]
                o1[pl.ds(i * 16, 16)] = res[1]
                o2[pl.ds(i * 16, 16)] = res[2]
                return c2

            lax.fori_loop(0, NG, group_b, 0)
            pltpu.sync_copy(o0, out.at[3 * b, pl.ds(off, P)])
            pltpu.sync_copy(o1, out.at[3 * b + 1, pl.ds(off, P)])
            pltpu.sync_copy(o2, out.at[3 * b + 2, pl.ds(off, P)])
            return carry

        lax.fori_loop(0, NCHUNK, chunk_body, 0)

    return warp


_interleave = _make_interleave()
_warp = _make_warp()


@jax.jit
def kernel(image, flow):
    img3 = image.reshape(B * C, H, W)
    table = _interleave(img3)
    flow2 = flow.reshape(B * 2, HW)
    out = _warp(table, flow2)
    return out.reshape(B, C, H, W)


# speedup vs baseline: 7.7704x; 1.6952x over previous
import functools

import jax
import jax.numpy as jnp
from jax import lax
from jax.experimental import pallas as pl
from jax.experimental.pallas import tpu as pltpu
from jax.experimental.pallas import tpu_sc as plsc

B, C, H, W = 16, 3, 512, 512
HW = H * W
W2 = W // 2
NW = 32
PIX_PER_TILE = B * HW // NW
TROWS = B * HW // 2 + 8

P = 1024
NCHUNK = PIX_PER_TILE // P
NG = P // 16
NIDX = 4 * P
NDMA = NIDX // 128

IR = 8
IPIX = IR * W
ICHUNK = 256 // IR

_MESH = plsc.VectorSubcoreMesh(core_axis_name="c", subcore_axis_name="s")
_CP = pltpu.CompilerParams(
    needs_layout_passes=False, use_tc_tiling_on_sc=False)


def _make_interleave():
    @functools.partial(
        pl.kernel,
        mesh=_MESH,
        out_type=jax.ShapeDtypeStruct((TROWS, 8), jnp.float32),
        compiler_params=_CP,
        scratch_types=[
            pltpu.VMEM((IR, W), jnp.float32),
            pltpu.VMEM((IR, W), jnp.float32),
            pltpu.VMEM((IR, W), jnp.float32),
            pltpu.VMEM((IPIX // 2, 8), jnp.float32),
        ],
    )
    def interleave(img, table, i0, i1, i2, buf):
        wid = lax.axis_index("s") * 2 + lax.axis_index("c")
        b = wid // 2
        half = wid % 2

        def chunk_body(ch, carry):
            r0 = half * 256 + ch * IR
            pltpu.sync_copy(img.at[3 * b, pl.ds(r0, IR), :], i0)
            pltpu.sync_copy(img.at[3 * b + 1, pl.ds(r0, IR), :], i1)
            pltpu.sync_copy(img.at[3 * b + 2, pl.ds(r0, IR), :], i2)

            def group(i, c2):
                rl = i // (W // 16)
                jb = (i % (W // 16)) * 16
                lane = lax.iota(jnp.int32, 16)
                px = jb + lane
                blk = rl * W2 + (px >> 1)
                col = (px & 1) << 2
                plsc.store_scatter(buf, [blk, col], i0[rl, pl.ds(jb, 16)])
                plsc.store_scatter(buf, [blk, col + 1], i1[rl, pl.ds(jb, 16)])
                plsc.store_scatter(buf, [blk, col + 2], i2[rl, pl.ds(jb, 16)])
                return c2

            lax.fori_loop(0, IPIX // 16, group, 0)
            pltpu.sync_copy(buf, table.at[pl.ds((b * H + r0) * W2, IPIX // 2)])
            return carry

        lax.fori_loop(0, ICHUNK, chunk_body, 0)

    return interleave


def _make_warp():
    buf_set = [
        pltpu.VMEM((P,), jnp.float32),
        pltpu.VMEM((P,), jnp.float32),
        pltpu.VMEM((P,), jnp.float32),
        pltpu.VMEM((P,), jnp.float32),
        pltpu.VMEM((P,), jnp.int32),
        pltpu.VMEM((NIDX,), jnp.int32),
        pltpu.VMEM((NIDX, 8), jnp.float32),
        pltpu.VMEM((P,), jnp.float32),
        pltpu.VMEM((P,), jnp.float32),
        pltpu.VMEM((P,), jnp.float32),
    ]

    @functools.partial(
        pl.kernel,
        mesh=_MESH,
        out_type=jax.ShapeDtypeStruct((B * C, HW), jnp.float32),
        compiler_params=_CP,
        scratch_types=buf_set + buf_set + [
            pltpu.SemaphoreType.DMA,
            pltpu.SemaphoreType.DMA,
            pltpu.SemaphoreType.DMA,
        ],
    )
    def warp(table, flow, out, *rest):
        bufs = (rest[0:10], rest[10:20])
        sem_g = (rest[20], rest[21])
        sem_o = rest[22]
        wid = lax.axis_index("s") * 2 + lax.axis_index("c")
        b = wid // 2
        half = wid % 2

        def pass_a(ch, par):
            fx_v, fy_v, wx_v, wy_v, ex_v, idx_v = bufs[par][:6]
            off = half * PIX_PER_TILE + lax.rem(ch, NCHUNK) * P
            row0 = off // W
            pltpu.sync_copy(flow.at[2 * b, pl.ds(off, P)], fx_v)
            pltpu.sync_copy(flow.at[2 * b + 1, pl.ds(off, P)], fy_v)

            def group_a(i, c2):
                r = row0 + i // (W // 16)
                jb = (i % (W // 16)) * 16
                lane = lax.iota(jnp.int32, 16)
                jf = (jb + lane).astype(jnp.float32)
                fx = fx_v[pl.ds(i * 16, 16)]
                fy = fy_v[pl.ds(i * 16, 16)]
                x = jnp.clip(jf - fx, 0.0, float(W - 1))
                y = jnp.clip(r.astype(jnp.float32) - fy, 0.0, float(H - 1))
                x0 = jnp.minimum(x.astype(jnp.int32), W - 2)
                y0 = jnp.minimum(y.astype(jnp.int32), H - 2)
                wx_v[pl.ds(i * 16, 16)] = x - x0.astype(jnp.float32)
                wy_v[pl.ds(i * 16, 16)] = y - y0.astype(jnp.float32)
                ex_v[pl.ds(i * 16, 16)] = (x0 & 1) << 2
                rt = (b * H + y0) * W2 + (x0 >> 1)
                rb = rt + W2
                pos = 4 * (i * 16 + lane)
                plsc.store_scatter(idx_v, [pos], rt)
                plsc.store_scatter(idx_v, [pos + 1], rt + 1)
                plsc.store_scatter(idx_v, [pos + 2], rb)
                plsc.store_scatter(idx_v, [pos + 3], rb + 1)
                return c2

            lax.fori_loop(0, NG, group_a, 0)

        def fire_gathers(par):
            idx_v, g_v = bufs[par][5], bufs[par][6]
            sem = sem_g[par]

            def dma_body(k, c2):
                for j in range(8):
                    d = k * 8 + j
                    pltpu.async_copy(
                        table.at[idx_v.at[pl.ds(d * 128, 128)]],
                        g_v.at[pl.ds(d * 128, 128)],
                        sem)
                return c2

            lax.fori_loop(0, NDMA // 8, dma_body, 0)

        def drain_gathers(par):
            g_v = bufs[par][6]
            sem = sem_g[par]

            def dma_body(k, c2):
                for j in range(8):
                    d = k * 8 + j
                    pltpu.make_async_copy(
                        table.at[pl.ds(0, 128)],
                        g_v.at[pl.ds(d * 128, 128)],
                        sem).wait()
                return c2

            lax.fori_loop(0, NDMA // 8, dma_body, 0)

        def pass_b(ch, par):
            wx_v, wy_v, ex_v, _, g_v, o0, o1, o2 = bufs[par][2:10]
            off = half * PIX_PER_TILE + ch * P

            def group_b(i, c2):
                lane = lax.iota(jnp.int32, 16)
                wx = wx_v[pl.ds(i * 16, 16)]
                wy = wy_v[pl.ds(i * 16, 16)]
                e = ex_v[pl.ds(i * 16, 16)]
                wxm = 1.0 - wx
                wym = 1.0 - wy
                p4 = 4 * (i * 16 + lane)
                res = []
                for c in range(3):
                    t0 = e + c
                    t4 = t0 + 4
                    ro = t4 >> 3
                    co = t4 & 7
                    v00 = plsc.load_gather(g_v, [p4, t0])
                    v01 = plsc.load_gather(g_v, [p4 + ro, co])
                    v10 = plsc.load_gather(g_v, [p4 + 2, t0])
                    v11 = plsc.load_gather(g_v, [p4 + 2 + ro, co])
                    res.append((v00 * wxm + v01 * wx) * wym
                               + (v10 * wxm + v11 * wx) * wy)
                o0[pl.ds(i * 16, 16)] = res[0]
                o1[pl.ds(i * 16, 16)] = res[1]
                o2[pl.ds(i * 16, 16)] = res[2]
                return c2

            lax.fori_loop(0, NG, group_b, 0)
            pltpu.async_copy(o0, out.at[3 * b, pl.ds(off, P)], sem_o)
            pltpu.async_copy(o1, out.at[3 * b + 1, pl.ds(off, P)], sem_o)
            pltpu.async_copy(o2, out.at[3 * b + 2, pl.ds(off, P)], sem_o)

        def drain_outs(n):
            for _ in range(n):
                pltpu.make_async_copy(
                    bufs[0][7], out.at[3 * b, pl.ds(0, P)], sem_o).wait()

        pass_a(0, 0)
        fire_gathers(0)

        def body(g, carry):
            pass_a(2 * g + 1, 1)
            fire_gathers(1)
            drain_gathers(0)
            pass_b(2 * g, 0)
            pass_a(2 * g + 2, 0)
            fire_gathers(0)
            drain_gathers(1)
            pass_b(2 * g + 1, 1)
            drain_outs(6)
            return carry

        lax.fori_loop(0, NCHUNK // 2, body, 0)
        drain_gathers(0)

    return warp


_interleave = _make_interleave()
_warp = _make_warp()


@jax.jit
def kernel(image, flow):
    img3 = image.reshape(B * C, H, W)
    table = _interleave(img3)
    flow2 = flow.reshape(B * 2, HW)
    out = _warp(table, flow2)
    return out.reshape(B, C, H, W)
